# baseline (speedup 1.0000x reference)
import functools

import jax
import jax.numpy as jnp
from jax import lax
from jax.experimental import pallas as pl
from jax.experimental.pallas import tpu as pltpu
from jax.experimental.pallas import tpu_sc as plsc

VOCAB = 1000000
D = 32
B = 16384
NF = 26
FT = 2 * NF
L = 16

NC = 2
NS = 16
NW = NC * NS
BPW = B // NW

BLK = 1024
NBLK = 976
NVB = BLK // L
TAIL = 512
NTT = 64
DP = 33

CHUNK = 2
ROWS = CHUNK * FT
NCH = BPW // CHUNK
GRPS = CHUNK * 2
INV = 1.0 / NF

_mesh = plsc.VectorSubcoreMesh(core_axis_name="c", subcore_axis_name="s")


@functools.partial(
    pl.kernel,
    mesh=_mesh,
    compiler_params=pltpu.CompilerParams(
        use_tc_tiling_on_sc=True, needs_layout_passes=False),
    out_type=jax.ShapeDtypeStruct((VOCAB * DP,), jnp.float32),
    scratch_types=[
        pltpu.VMEM((D, BLK), jnp.float32),
        pltpu.VMEM((D, BLK), jnp.float32),
        pltpu.VMEM((BLK * DP,), jnp.float32),
        pltpu.SemaphoreType.DMA,
        pltpu.SemaphoreType.DMA,
    ],
)
def _detranspose(tt_hbm, tail_hbm, dense_hbm, stage0, stage1, outb,
                 sin0, sin1):
    wid = lax.axis_index("s") * NC + lax.axis_index("c")
    base = wid * 30 + jnp.minimum(wid, 16)
    nblk = jnp.where(wid < 16, 31, 30)

    iota33 = lax.iota(jnp.int32, L) * DP
    stages = (stage0, stage1)
    sins = (sin0, sin1)

    def fetch(g, b):
        return pltpu.make_async_copy(
            tt_hbm.at[:, pl.ds((base + g) * BLK, BLK)], stages[b], sins[b])

    fetch(0, 0).start()
    fetch(1, 1).start()

    def transpose_block(stage, nv, i0):
        @plsc.parallel_loop(0, nv, unroll=2)
        def vgroup(v):
            av = iota33 + v * (L * DP)
            for d in range(D):
                x = stage[d, pl.ds(v * L, L)]
                plsc.store_scatter(outb, [av + d], x)
        pltpu.sync_copy(
            outb.at[pl.ds(0, nv * L * DP)],
            dense_hbm.at[pl.ds(i0 * DP, nv * L * DP)])

    def body(i, carry):
        for b in range(2):
            g = 2 * i + b

            @pl.when(g < nblk)
            def _():
                fetch(g, b).wait()
                transpose_block(stages[b], NVB, (base + g) * BLK)

                @pl.when(g + 2 < nblk)
                def _():
                    fetch(g + 2, b).start()

        return carry

    lax.fori_loop(0, 16, body, 0)

    @pl.when(wid == 31)
    def _():
        pltpu.sync_copy(
            tt_hbm.at[:, pl.ds(NBLK * BLK, TAIL)],
            stage0.at[:, pl.ds(0, TAIL)])
        transpose_block(stage0, TAIL // L, NBLK * BLK)

    @pl.when(wid == 30)
    def _():
        pltpu.sync_copy(tail_hbm, outb.at[pl.ds(0, NTT * DP)])
        pltpu.sync_copy(outb.at[pl.ds(0, NTT * DP)],
                        dense_hbm.at[pl.ds((VOCAB - NTT) * DP, NTT * DP)])


@functools.partial(
    pl.kernel,
    mesh=_mesh,
    compiler_params=pltpu.CompilerParams(use_tc_tiling_on_sc=False),
    out_type=jax.ShapeDtypeStruct((B * 2, D), jnp.float32),
    scratch_types=[
        pltpu.VMEM((NCH, ROWS), jnp.int32),
        pltpu.VMEM((ROWS, DP), jnp.float32),
        pltpu.VMEM((ROWS, DP), jnp.float32),
        pltpu.VMEM((BPW * 2, D), jnp.float32),
        pltpu.SemaphoreType.DMA,
        pltpu.SemaphoreType.DMA,
    ],
)
def _emb_pool(idx_hbm, table_hbm, out_hbm, idx_v, rows0, rows1, out_v,
              sem0, sem1):
    wid = lax.axis_index("s") * NC + lax.axis_index("c")
    base = wid * NCH

    pltpu.sync_copy(idx_hbm.at[pl.ds(base, NCH)], idx_v)

    bufs = (rows0, rows1)
    sems = (sem0, sem1)

    def gather(g, b, sem):
        return pltpu.make_async_copy(table_hbm.at[idx_v.at[g]], bufs[b], sem)

    gather(0, 0, sem0).start()
    gather(1, 1, sem1).start()

    def reduce_chunk(g, buf):
        for grp in range(GRPS):
            s = grp * NF
            a0 = buf[s, 0:L]
            a1 = buf[s, L:D]
            for j in range(1, NF):
                a0 = a0 + buf[s + j, 0:L]
                a1 = a1 + buf[s + j, L:D]
            orow = g * GRPS + grp
            out_v[orow, 0:L] = a0 * INV
            out_v[orow, L:D] = a1 * INV

    def body(i, carry):
        for b in range(2):
            g = 2 * i + b
            gather(g, b, sems[b]).wait()
            reduce_chunk(g, bufs[b])

            @pl.when(g < NCH - 2)
            def _():
                gather(g + 2, b, sems[b]).start()

        return carry

    lax.fori_loop(0, NCH // 2, body, 0)

    pltpu.sync_copy(out_v, out_hbm.at[pl.ds(base * GRPS, BPW * 2)])


def kernel(x1, x2, table):
    idx = jnp.concatenate(
        [x1.astype(jnp.int32), x2.astype(jnp.int32)], axis=1)
    idx = idx.reshape(B // CHUNK, ROWS)
    tail = jnp.pad(table[VOCAB - NTT:], ((0, 0), (0, DP - D)))
    tail = tail.reshape(NTT * DP)
    dense = _detranspose(table.T, tail)
    out = _emb_pool(idx, dense.reshape(VOCAB, DP))
    return out.reshape(B, 2, D)


# speedup vs baseline: 4.7762x; 4.7762x over previous
import functools

import jax
import jax.numpy as jnp
from jax import lax
from jax.experimental import pallas as pl
from jax.experimental.pallas import tpu as pltpu
from jax.experimental.pallas import tpu_sc as plsc

VOCAB = 1000000
D = 32
B = 16384
NF = 26
FT = 2 * NF
L = 16

NC = 2
NS = 16
NW = NC * NS
BPW = B // NW

BLK = 512
NBLK = 1953
NVB = BLK // L
NTT = 64
DP = 33

CHUNK = 2
ROWS = CHUNK * FT
NCH = BPW // CHUNK
GRPS = CHUNK * 2
INV = 1.0 / NF

_mesh = plsc.VectorSubcoreMesh(core_axis_name="c", subcore_axis_name="s")


@functools.partial(
    pl.kernel,
    mesh=_mesh,
    compiler_params=pltpu.CompilerParams(
        use_tc_tiling_on_sc=True, needs_layout_passes=False),
    out_type=jax.ShapeDtypeStruct((VOCAB * D,), jnp.float32),
    scratch_types=[
        pltpu.VMEM((D, BLK), jnp.float32),
        pltpu.VMEM((D, BLK), jnp.float32),
        pltpu.VMEM((BLK * DP,), jnp.float32),
        pltpu.VMEM((BLK * D,), jnp.float32),
        pltpu.SemaphoreType.DMA,
        pltpu.SemaphoreType.DMA,
    ],
)
def _detranspose(tt_hbm, tail_hbm, dense_hbm, stage0, stage1, outb, outc,
                 sin0, sin1):
    wid = lax.axis_index("s") * NC + lax.axis_index("c")
    base = wid * 61
    nblk = jnp.where(wid == 31, 62, 61)

    iota33 = lax.iota(jnp.int32, L) * DP
    stages = (stage0, stage1)
    sins = (sin0, sin1)

    def fetch(g, b):
        return pltpu.make_async_copy(
            tt_hbm.at[:, pl.ds((base + g) * BLK, BLK)], stages[b], sins[b])

    fetch(0, 0).start()
    fetch(1, 1).start()

    def transpose_block(stage, nv, i0):
        @plsc.parallel_loop(0, nv, unroll=2)
        def vgroup(v):
            av = iota33 + v * (L * DP)
            for d in range(D):
                x = stage[d, pl.ds(v * L, L)]
                plsc.store_scatter(outb, [av + d], x)

        @plsc.parallel_loop(0, nv * L, unroll=4)
        def rowcpy(r):
            outc[pl.ds(r * D, L)] = outb[pl.ds(r * DP, L)]
            outc[pl.ds(r * D + L, L)] = outb[pl.ds(r * DP + L, L)]

        pltpu.sync_copy(
            outc.at[pl.ds(0, nv * L * D)],
            dense_hbm.at[pl.ds(i0 * D, nv * L * D)])

    def body(i, carry):
        for b in range(2):
            g = 2 * i + b

            @pl.when(g < nblk)
            def _():
                fetch(g, b).wait()
                transpose_block(stages[b], NVB, (base + g) * BLK)

                @pl.when(g + 2 < nblk)
                def _():
                    fetch(g + 2, b).start()

        return carry

    lax.fori_loop(0, 31, body, 0)

    @pl.when(wid == 30)
    def _():
        pltpu.sync_copy(
            tail_hbm, dense_hbm.at[pl.ds((VOCAB - NTT) * D, NTT * D)])


@functools.partial(
    pl.kernel,
    mesh=_mesh,
    compiler_params=pltpu.CompilerParams(use_tc_tiling_on_sc=False),
    out_type=jax.ShapeDtypeStruct((B * 2, D), jnp.float32),
    scratch_types=[
        pltpu.VMEM((NCH, ROWS), jnp.int32),
        pltpu.VMEM((ROWS, D), jnp.float32),
        pltpu.VMEM((ROWS, D), jnp.float32),
        pltpu.VMEM((BPW * 2, D), jnp.float32),
        pltpu.SemaphoreType.DMA,
        pltpu.SemaphoreType.DMA,
    ],
)
def _emb_pool(idx_hbm, table_hbm, out_hbm, idx_v, rows0, rows1, out_v,
              sem0, sem1):
    wid = lax.axis_index("s") * NC + lax.axis_index("c")
    base = wid * NCH

    pltpu.sync_copy(idx_hbm.at[pl.ds(base, NCH)], idx_v)

    bufs = (rows0, rows1)
    sems = (sem0, sem1)

    def gather(g, b, sem):
        return pltpu.make_async_copy(table_hbm.at[idx_v.at[g]], bufs[b], sem)

    gather(0, 0, sem0).start()
    gather(1, 1, sem1).start()

    def reduce_chunk(g, buf):
        for grp in range(GRPS):
            s = grp * NF
            a0 = buf[s, 0:L]
            a1 = buf[s, L:D]
            for j in range(1, NF):
                a0 = a0 + buf[s + j, 0:L]
                a1 = a1 + buf[s + j, L:D]
            orow = g * GRPS + grp
            out_v[orow, 0:L] = a0 * INV
            out_v[orow, L:D] = a1 * INV

    def body(i, carry):
        for b in range(2):
            g = 2 * i + b
            gather(g, b, sems[b]).wait()
            reduce_chunk(g, bufs[b])

            @pl.when(g < NCH - 2)
            def _():
                gather(g + 2, b, sems[b]).start()

        return carry

    lax.fori_loop(0, NCH // 2, body, 0)

    pltpu.sync_copy(out_v, out_hbm.at[pl.ds(base * GRPS, BPW * 2)])


def kernel(x1, x2, table):
    idx = jnp.concatenate(
        [x1.astype(jnp.int32), x2.astype(jnp.int32)], axis=1)
    idx = idx.reshape(B // CHUNK, ROWS)
    tail = table[VOCAB - NTT:].reshape(NTT * D)
    dense = _detranspose(table.T, tail)
    out = _emb_pool(idx, dense.reshape(VOCAB, D))
    return out.reshape(B, 2, D)


# speedup vs baseline: 5.6373x; 1.1803x over previous
import functools

import jax
import jax.numpy as jnp
from jax import lax
from jax.experimental import pallas as pl
from jax.experimental.pallas import tpu as pltpu
from jax.experimental.pallas import tpu_sc as plsc

VOCAB = 1000000
D = 32
B = 16384
NF = 26
FT = 2 * NF
L = 16

NC = 2
NS = 16
NW = NC * NS
BPW = B // NW

BLK = 512
NBLK = 1953
NVB = BLK // L
NTT = 64
DP = 33

CHUNK = 2
ROWS = CHUNK * FT
NCH = BPW // CHUNK
GRPS = CHUNK * 2
INV = 1.0 / NF

_mesh = plsc.VectorSubcoreMesh(core_axis_name="c", subcore_axis_name="s")


@functools.partial(
    pl.kernel,
    mesh=_mesh,
    compiler_params=pltpu.CompilerParams(
        use_tc_tiling_on_sc=True, needs_layout_passes=False),
    out_type=jax.ShapeDtypeStruct((VOCAB * D,), jnp.float32),
    scratch_types=[
        pltpu.VMEM((D, BLK), jnp.float32),
        pltpu.VMEM((D, BLK), jnp.float32),
        pltpu.VMEM((BLK * DP,), jnp.float32),
        pltpu.VMEM((BLK * D,), jnp.float32),
        pltpu.SemaphoreType.DMA,
        pltpu.SemaphoreType.DMA,
    ],
)
def _detranspose(tt_hbm, tail_hbm, dense_hbm, stage0, stage1, outb, outc,
                 sin0, sin1):
    wid = lax.axis_index("s") * NC + lax.axis_index("c")
    base = wid * 61
    nblk = jnp.where(wid == 31, 62, 61)

    iota33 = lax.iota(jnp.int32, L) * DP
    stages = (stage0, stage1)
    sins = (sin0, sin1)

    def fetch(g, b):
        return pltpu.make_async_copy(
            tt_hbm.at[:, pl.ds((base + g) * BLK, BLK)], stages[b], sins[b])

    fetch(0, 0).start()
    fetch(1, 1).start()

    def transpose_block(stage, nv, i0):
        @plsc.parallel_loop(0, nv, unroll=4)
        def vgroup(v):
            av = iota33 + v * (L * DP)
            for d in range(D):
                x = stage[d, pl.ds(v * L, L)]
                plsc.store_scatter(outb, [av + d], x)

        @plsc.parallel_loop(0, nv * L, unroll=8)
        def rowcpy(r):
            outc[pl.ds(r * D, L)] = outb[pl.ds(r * DP, L)]
            outc[pl.ds(r * D + L, L)] = outb[pl.ds(r * DP + L, L)]

        pltpu.sync_copy(
            outc.at[pl.ds(0, nv * L * D)],
            dense_hbm.at[pl.ds(i0 * D, nv * L * D)])

    def body(i, carry):
        for b in range(2):
            g = 2 * i + b

            @pl.when(g < nblk)
            def _():
                fetch(g, b).wait()
                transpose_block(stages[b], NVB, (base + g) * BLK)

                @pl.when(g + 2 < nblk)
                def _():
                    fetch(g + 2, b).start()

        return carry

    lax.fori_loop(0, 31, body, 0)

    @pl.when(wid == 30)
    def _():
        pltpu.sync_copy(
            tail_hbm, dense_hbm.at[pl.ds((VOCAB - NTT) * D, NTT * D)])


@functools.partial(
    pl.kernel,
    mesh=_mesh,
    compiler_params=pltpu.CompilerParams(use_tc_tiling_on_sc=False),
    out_type=jax.ShapeDtypeStruct((B * 2, D), jnp.float32),
    scratch_types=[
        pltpu.VMEM((NCH, ROWS), jnp.int32),
        pltpu.VMEM((ROWS, D), jnp.float32),
        pltpu.VMEM((ROWS, D), jnp.float32),
        pltpu.VMEM((ROWS, D), jnp.float32),
        pltpu.VMEM((ROWS, D), jnp.float32),
        pltpu.VMEM((BPW * 2, D), jnp.float32),
        pltpu.SemaphoreType.DMA,
        pltpu.SemaphoreType.DMA,
        pltpu.SemaphoreType.DMA,
        pltpu.SemaphoreType.DMA,
    ],
)
def _emb_pool(idx_hbm, table_hbm, out_hbm, idx_v, rows0, rows1, rows2, rows3,
              out_v, sem0, sem1, sem2, sem3):
    wid = lax.axis_index("s") * NC + lax.axis_index("c")
    base = wid * NCH

    pltpu.sync_copy(idx_hbm.at[pl.ds(base, NCH)], idx_v)

    bufs = (rows0, rows1, rows2, rows3)
    sems = (sem0, sem1, sem2, sem3)

    def gather(g, b):
        return pltpu.make_async_copy(table_hbm.at[idx_v.at[g]], bufs[b],
                                     sems[b])

    for b in range(4):
        gather(b, b).start()

    def reduce_chunk(g, buf):
        for grp in range(GRPS):
            s = grp * NF
            a0 = buf[s, 0:L]
            a1 = buf[s, L:D]
            for j in range(1, NF):
                a0 = a0 + buf[s + j, 0:L]
                a1 = a1 + buf[s + j, L:D]
            orow = g * GRPS + grp
            out_v[orow, 0:L] = a0 * INV
            out_v[orow, L:D] = a1 * INV

    def body(i, carry):
        for b in range(4):
            g = 4 * i + b
            gather(g, b).wait()
            reduce_chunk(g, bufs[b])

            @pl.when(g < NCH - 4)
            def _():
                gather(g + 4, b).start()

        return carry

    lax.fori_loop(0, NCH // 4, body, 0)

    pltpu.sync_copy(out_v, out_hbm.at[pl.ds(base * GRPS, BPW * 2)])


def kernel(x1, x2, table):
    idx = jnp.concatenate(
        [x1.astype(jnp.int32), x2.astype(jnp.int32)], axis=1)
    idx = idx.reshape(B // CHUNK, ROWS)
    tail = table[VOCAB - NTT:].reshape(NTT * D)
    dense = _detranspose(table.T, tail)
    out = _emb_pool(idx, dense.reshape(VOCAB, D))
    return out.reshape(B, 2, D)


# speedup vs baseline: 6.1744x; 1.0953x over previous
import functools

import jax
import jax.numpy as jnp
from jax import lax
from jax.experimental import pallas as pl
from jax.experimental.pallas import tpu as pltpu
from jax.experimental.pallas import tpu_sc as plsc

VOCAB = 1000000
D = 32
B = 16384
NF = 26
FT = 2 * NF
L = 16

NC = 2
NS = 16
NW = NC * NS
BPW = B // NW

BLK = 512
NBLK = 1953
NVB = BLK // L
NTT = 64
DP = 33

CHUNK = 2
ROWS = CHUNK * FT
NCH = BPW // CHUNK
GRPS = CHUNK * 2
INV = 1.0 / NF

_mesh = plsc.VectorSubcoreMesh(core_axis_name="c", subcore_axis_name="s")


@functools.partial(
    pl.kernel,
    mesh=_mesh,
    compiler_params=pltpu.CompilerParams(
        use_tc_tiling_on_sc=True, needs_layout_passes=False),
    out_type=jax.ShapeDtypeStruct((VOCAB * D,), jnp.float32),
    scratch_types=[
        pltpu.VMEM((D, BLK), jnp.float32),
        pltpu.VMEM((D, BLK), jnp.float32),
        pltpu.VMEM((BLK * DP,), jnp.float32),
        pltpu.VMEM((BLK * D,), jnp.float32),
        pltpu.SemaphoreType.DMA,
        pltpu.SemaphoreType.DMA,
        pltpu.SemaphoreType.DMA,
    ],
)
def _detranspose(tt_hbm, tail_hbm, dense_hbm, stage0, stage1, outb, outc,
                 sin0, sin1, sodma):
    wid = lax.axis_index("s") * NC + lax.axis_index("c")
    base = wid * 61
    nblk = jnp.where(wid == 31, 62, 61)

    iota33 = lax.iota(jnp.int32, L) * DP
    stages = (stage0, stage1)
    sins = (sin0, sin1)

    def fetch(g, b):
        return pltpu.make_async_copy(
            tt_hbm.at[:, pl.ds((base + g) * BLK, BLK)], stages[b], sins[b])

    fetch(0, 0).start()
    fetch(1, 1).start()

    def outdma(nv, i0):
        return pltpu.make_async_copy(
            outc.at[pl.ds(0, nv * L * D)],
            dense_hbm.at[pl.ds(i0 * D, nv * L * D)], sodma)

    def transpose_block(stage, nv, i0, first):
        @plsc.parallel_loop(0, nv, unroll=4)
        def vgroup(v):
            av = iota33 + v * (L * DP)
            for d in range(D):
                x = stage[d, pl.ds(v * L, L)]
                plsc.store_scatter(outb, [av + d], x)

        @pl.when(jnp.logical_not(first))
        def _():
            outdma(nv, i0).wait()

        @plsc.parallel_loop(0, nv * L, unroll=8)
        def rowcpy(r):
            outc[pl.ds(r * D, L)] = outb[pl.ds(r * DP, L)]
            outc[pl.ds(r * D + L, L)] = outb[pl.ds(r * DP + L, L)]

        outdma(nv, i0).start()

    def body(i, carry):
        for b in range(2):
            g = 2 * i + b

            @pl.when(g < nblk)
            def _():
                fetch(g, b).wait()
                transpose_block(stages[b], NVB, (base + g) * BLK, g == 0)

                @pl.when(g + 2 < nblk)
                def _():
                    fetch(g + 2, b).start()

        return carry

    lax.fori_loop(0, 31, body, 0)
    outdma(NVB, base * BLK).wait()

    @pl.when(wid == 30)
    def _():
        pltpu.sync_copy(
            tail_hbm, dense_hbm.at[pl.ds((VOCAB - NTT) * D, NTT * D)])


@functools.partial(
    pl.kernel,
    mesh=_mesh,
    compiler_params=pltpu.CompilerParams(use_tc_tiling_on_sc=False),
    out_type=jax.ShapeDtypeStruct((B * 2, D), jnp.float32),
    scratch_types=[
        pltpu.VMEM((NCH, ROWS), jnp.int32),
        pltpu.VMEM((ROWS, D), jnp.float32),
        pltpu.VMEM((ROWS, D), jnp.float32),
        pltpu.VMEM((ROWS, D), jnp.float32),
        pltpu.VMEM((ROWS, D), jnp.float32),
        pltpu.VMEM((BPW * 2, D), jnp.float32),
        pltpu.SemaphoreType.DMA,
        pltpu.SemaphoreType.DMA,
        pltpu.SemaphoreType.DMA,
        pltpu.SemaphoreType.DMA,
    ],
)
def _emb_pool(idx_hbm, table_hbm, out_hbm, idx_v, rows0, rows1, rows2, rows3,
              out_v, sem0, sem1, sem2, sem3):
    wid = lax.axis_index("s") * NC + lax.axis_index("c")
    base = wid * NCH

    pltpu.sync_copy(idx_hbm.at[pl.ds(base, NCH)], idx_v)

    bufs = (rows0, rows1, rows2, rows3)
    sems = (sem0, sem1, sem2, sem3)

    def gather(g, b):
        return pltpu.make_async_copy(table_hbm.at[idx_v.at[g]], bufs[b],
                                     sems[b])

    for b in range(4):
        gather(b, b).start()

    def reduce_chunk(g, buf):
        for grp in range(GRPS):
            s = grp * NF
            a0 = buf[s, 0:L]
            a1 = buf[s, L:D]
            for j in range(1, NF):
                a0 = a0 + buf[s + j, 0:L]
                a1 = a1 + buf[s + j, L:D]
            orow = g * GRPS + grp
            out_v[orow, 0:L] = a0 * INV
            out_v[orow, L:D] = a1 * INV

    def body(i, carry):
        for b in range(4):
            g = 4 * i + b
            gather(g, b).wait()
            reduce_chunk(g, bufs[b])

            @pl.when(g < NCH - 4)
            def _():
                gather(g + 4, b).start()

        return carry

    lax.fori_loop(0, NCH // 4, body, 0)

    pltpu.sync_copy(out_v, out_hbm.at[pl.ds(base * GRPS, BPW * 2)])


def kernel(x1, x2, table):
    idx = jnp.concatenate(
        [x1.astype(jnp.int32), x2.astype(jnp.int32)], axis=1)
    idx = idx.reshape(B // CHUNK, ROWS)
    tail = table[VOCAB - NTT:].reshape(NTT * D)
    dense = _detranspose(table.T, tail)
    out = _emb_pool(idx, dense.reshape(VOCAB, D))
    return out.reshape(B, 2, D)
